# baseline (speedup 1.0000x reference)
import functools

import jax
import jax.numpy as jnp
from jax import lax
from jax.experimental import pallas as pl
from jax.experimental.pallas import tpu as pltpu
from jax.experimental.pallas import tpu_sc as plsc

N = 10000
F = 128
B = 64
NC = 2
NS = 16
NW = NC * NS
L = 16
CH = 320
TAIL_START = 31 * CH
TAIL = N - TAIL_START
HALF = CH // 2
NJ = F // L


def _sc_segment_partials(x, batch):
    mesh = plsc.VectorSubcoreMesh(core_axis_name="c", subcore_axis_name="s")

    SL = B * F // NS
    SLC = B * L // NS

    @functools.partial(
        pl.kernel,
        out_type=[
            jax.ShapeDtypeStruct((NC * B, F), jnp.float32),
            jax.ShapeDtypeStruct((NC * B * L // F, F), jnp.float32),
        ],
        mesh=mesh,
        scratch_types=[
            pltpu.VMEM((HALF, F), jnp.float32),
            pltpu.VMEM((HALF, F), jnp.float32),
            pltpu.VMEM((CH,), jnp.int32),
            pltpu.VMEM((B * F,), jnp.float32),
            pltpu.VMEM((B * L,), jnp.float32),
            pltpu.VMEM((NS, SL), jnp.float32),
            pltpu.VMEM((NS, SLC), jnp.float32),
            pltpu.VMEM_SHARED((NS, B * F), jnp.float32),
            pltpu.VMEM_SHARED((NS, B * L), jnp.float32),
            pltpu.SemaphoreType.DMA,
            pltpu.SemaphoreType.DMA,
        ],
    )
    def sc_kernel(x_hbm, b_hbm, psum_hbm, pcnt_hbm, xa, xb, bv, acc, cnt,
                  rsum, rcnt, sh_sum, sh_cnt, sa, sb):
        sid = lax.axis_index("s")
        cid = lax.axis_index("c")
        wid = sid * NC + cid

        zeros = jnp.zeros((L,), jnp.float32)
        ones_v = jnp.ones((L,), jnp.float32)

        def flush(seg, accv, cntf):
            plsc.addupdate(cnt.at[pl.ds(seg * L, L)], cntf)
            for j in range(NJ):
                plsc.addupdate(acc.at[pl.ds(seg * F + j * L, L)], accv[j])

        def chunk_groups(xv, roff_g, lo, hi, carry):
            def grp_body(g, c):
                cur, dirty, cntf, accv = c
                segv = bv[pl.ds(g * L, L)]
                sk = [segv[k] for k in range(L)]
                s0 = sk[0]
                s15 = sk[L - 1]
                bts = [sk[t] == s0 for t in range(1, L)]
                pf = jnp.float32(1.0)
                for t in range(L - 1):
                    pf = pf + jnp.where(bts[t], 1.0, 0.0)
                interior = jnp.float32(0.0)
                for t in range(1, L - 1):
                    bad = jnp.logical_and(sk[t] != s0, sk[t] != s15)
                    interior = jnp.maximum(interior,
                                           jnp.where(bad, 1.0, 0.0))
                um = 1.0 - interior
                k1 = jnp.where(s0 == cur, 1.0, 0.0)
                k2 = jnp.where(s15 == s0, 1.0, 0.0)

                glo = []
                full = []
                for j in range(NJ):
                    p = xv[(g + roff_g) * L, pl.ds(j * L, L)]
                    sel = p
                    for t in range(1, L):
                        p = p + xv[(g + roff_g) * L + t, pl.ds(j * L, L)]
                        sel = jnp.where(bts[t - 1], p, sel)
                    full.append(p)
                    glo.append(sel)

                @pl.when(jnp.logical_or(s0 != cur, interior > 0.5))
                def _():
                    flush(cur, accv, cntf)

                after_lo = [accv[j] * k1 + glo[j] * um for j in range(NJ)]
                cnt_lo = cntf * k1 + ones_v * (pf * um)

                @pl.when(jnp.logical_and(s15 != s0, interior < 0.5))
                def _():
                    flush(s0, after_lo, cnt_lo)

                accv = [(after_lo[j] * k2 +
                         (full[j] - glo[j]) * (1.0 - k2)) * um
                        for j in range(NJ)]
                cntf = (cnt_lo * k2 +
                        ones_v * ((16.0 - pf) * (1.0 - k2))) * um
                dirty = jnp.maximum(dirty, interior)
                return (s15, dirty, cntf, accv)

            return pl.loop(lo, hi, init_carry=carry)(grp_body)

        def slow_chunk(xv, roff_g, lo, hi):
            def sbody(g):
                segv = bv[pl.ds(g * L, L)]
                sk = [segv[k] for k in range(L)]
                interior = jnp.float32(0.0)
                for t in range(1, L - 1):
                    bad = jnp.logical_and(sk[t] != sk[0],
                                          sk[t] != sk[L - 1])
                    interior = jnp.maximum(interior,
                                           jnp.where(bad, 1.0, 0.0))

                @pl.when(interior > 0.5)
                def _():
                    for k in range(L):
                        s = sk[k]
                        plsc.addupdate(cnt.at[pl.ds(s * L, L)], ones_v)
                        for j in range(NJ):
                            plsc.addupdate(
                                acc.at[pl.ds(s * F + j * L, L)],
                                xv[(g + roff_g) * L + k, pl.ds(j * L, L)])

            pl.loop(lo, hi)(sbody)

        def zero_acc():
            def zero_body(i):
                for j in range(NJ):
                    acc[pl.ds(i * F + j * L, L)] = zeros
                cnt[pl.ds(i * L, L)] = zeros

            pl.loop(0, B)(zero_body)

        start = jnp.where(wid == NW - 1, N - CH, wid * CH)
        glo = jnp.where(wid == NW - 1, (CH - TAIL) // L, 0)
        c0 = pltpu.async_copy(x_hbm.at[pl.ds(start, HALF)], xa, sa)
        c1 = pltpu.async_copy(x_hbm.at[pl.ds(start + HALF, HALF)], xb, sb)
        pltpu.sync_copy(b_hbm.at[pl.ds(start, CH)], bv)
        zero_acc()
        sglo = bv[pl.ds(glo * L, L)][0]
        carry = (sglo, jnp.float32(0.0), zeros, [zeros for _ in range(NJ)])
        c0.wait()
        carry = chunk_groups(xa, 0, glo, HALF // L, carry)
        c1.wait()
        carry = chunk_groups(xb, -(HALF // L), jnp.maximum(glo, HALF // L),
                             CH // L, carry)
        flush(carry[0], carry[3], carry[2])

        @pl.when(carry[1] > 0.5)
        def _():
            slow_chunk(xa, 0, glo, HALF // L)
            slow_chunk(xb, -(HALF // L), jnp.maximum(glo, HALF // L),
                       CH // L)

        p0 = pltpu.async_copy(acc, sh_sum.at[sid], sa)
        p1 = pltpu.async_copy(cnt, sh_cnt.at[sid], sb)
        p0.wait()
        p1.wait()
        plsc.subcore_barrier()
        cps = []
        for r in range(NS):
            cps.append(pltpu.async_copy(
                sh_sum.at[r, pl.ds(sid * SL, SL)], rsum.at[r], sa))
            cps.append(pltpu.async_copy(
                sh_cnt.at[r, pl.ds(sid * SLC, SLC)], rcnt.at[r], sb))
        for c in cps:
            c.wait()

        def red_sum(m):
            tot = rsum[0, pl.ds(m * L, L)]
            for r in range(1, NS):
                tot = tot + rsum[r, pl.ds(m * L, L)]
            acc[pl.ds(m * L, L)] = tot

        pl.loop(0, SL // L)(red_sum)

        for m in range(SLC // L):
            tot = rcnt[0, pl.ds(m * L, L)]
            for r in range(1, NS):
                tot = tot + rcnt[r, pl.ds(m * L, L)]
            cnt[pl.ds(m * L, L)] = tot

        RPT = B // NS
        wb = []
        for r in range(RPT):
            wb.append(pltpu.async_copy(
                acc.at[pl.ds(r * F, F)],
                psum_hbm.at[cid * B + sid * RPT + r], sa))
        wb.append(pltpu.async_copy(
            cnt.at[pl.ds(0, SLC)],
            pcnt_hbm.at[cid * 8 + sid // 2, pl.ds((sid % 2) * SLC, SLC)],
            sb))
        for c in wb:
            c.wait()

    return sc_kernel(x, batch)


def _tc_finish(psum, pcnt, u, W, b2):
    def tc_body(ps_ref, pc_ref, u_ref, w_ref, b_ref, out_ref):
        ps = ps_ref[...]
        sums = ps[:B] + ps[B:]
        ct = pc_ref[...]
        ctot = ct[:8] + ct[8:]
        si = jax.lax.broadcasted_iota(jnp.int32, (B, 8), 0)
        ri = jax.lax.broadcasted_iota(jnp.int32, (B, 8), 1)
        rsel = (ri == si // 8).astype(jnp.float32)
        rows = jnp.dot(rsel, ctot, preferred_element_type=jnp.float32)
        si2 = jax.lax.broadcasted_iota(jnp.int32, (B, F), 0)
        ci = jax.lax.broadcasted_iota(jnp.int32, (B, F), 1)
        csel = (ci == (si2 % 8) * L).astype(jnp.float32)
        counts = jnp.sum(rows * csel, axis=1, keepdims=True)
        x_agg = sums / jnp.maximum(counts, 1.0)
        w = w_ref[...]
        out = (
            jnp.dot(x_agg, w[:F], preferred_element_type=jnp.float32)
            + jnp.dot(u_ref[...], w[F:], preferred_element_type=jnp.float32)
            + b_ref[...]
        )
        out_ref[...] = out

    return pl.pallas_call(
        tc_body,
        out_shape=jax.ShapeDtypeStruct((B, B), jnp.float32),
    )(psum, pcnt, u, W, b2)


def kernel(x, edge_index, edge_attr, u, batch, W, b):
    psum, pcnt = _sc_segment_partials(x, batch)
    return _tc_finish(psum, pcnt, u, W, b.reshape(1, B))


# speedup vs baseline: 1.0385x; 1.0385x over previous
import functools

import jax
import jax.numpy as jnp
from jax import lax
from jax.experimental import pallas as pl
from jax.experimental.pallas import tpu as pltpu
from jax.experimental.pallas import tpu_sc as plsc

N = 10000
F = 128
B = 64
NC = 2
NS = 16
NW = NC * NS
L = 16
CH = 320
TAIL_START = 31 * CH
TAIL = N - TAIL_START
HALF = CH // 2
NJ = F // L


def _sc_segment_partials(x, batch):
    mesh = plsc.VectorSubcoreMesh(core_axis_name="c", subcore_axis_name="s")

    SL = B * F // NS
    SLC = B * L // NS

    @functools.partial(
        pl.kernel,
        out_type=[
            jax.ShapeDtypeStruct((NC * B, F), jnp.float32),
            jax.ShapeDtypeStruct((NC * B * L // F, F), jnp.float32),
        ],
        mesh=mesh,
        scratch_types=[
            pltpu.VMEM((HALF, F), jnp.float32),
            pltpu.VMEM((HALF, F), jnp.float32),
            pltpu.VMEM((CH,), jnp.int32),
            pltpu.VMEM((B * F,), jnp.float32),
            pltpu.VMEM((B * L,), jnp.float32),
            pltpu.VMEM((NS, SL), jnp.float32),
            pltpu.VMEM((NS, SLC), jnp.float32),
            pltpu.VMEM_SHARED((NS, B * F), jnp.float32),
            pltpu.VMEM_SHARED((NS, B * L), jnp.float32),
            pltpu.SemaphoreType.DMA,
            pltpu.SemaphoreType.DMA,
        ],
    )
    def sc_kernel(x_hbm, b_hbm, psum_hbm, pcnt_hbm, xa, xb, bv, acc, cnt,
                  rsum, rcnt, sh_sum, sh_cnt, sa, sb):
        sid = lax.axis_index("s")
        cid = lax.axis_index("c")
        wid = sid * NC + cid

        zeros = jnp.zeros((L,), jnp.float32)
        ones_v = jnp.ones((L,), jnp.float32)

        def flush(seg, accv, cntf):
            plsc.addupdate(cnt.at[pl.ds(seg * L, L)], cntf)
            for j in range(NJ):
                plsc.addupdate(acc.at[pl.ds(seg * F + j * L, L)], accv[j])

        def chunk_groups(xv, roff_g, lo, hi, carry):
            def grp_body(g, c):
                cur, cntf, accv = c
                segv = bv[pl.ds(g * L, L)]
                for k in range(L):
                    s = segv[k]
                    is_new = s != cur

                    @pl.when(is_new)
                    def _():
                        flush(cur, accv, cntf)

                    keep = jnp.where(is_new, 0.0, 1.0)
                    row = [xv[(g + roff_g) * L + k, pl.ds(j * L, L)]
                           for j in range(NJ)]
                    accv = [accv[j] * keep + row[j] for j in range(NJ)]
                    cntf = cntf * keep + ones_v
                    cur = s
                return (cur, cntf, accv)

            return pl.loop(lo, hi, init_carry=carry)(grp_body)

        def zero_acc():
            def zero_body(i):
                for j in range(NJ):
                    acc[pl.ds(i * F + j * L, L)] = zeros
                cnt[pl.ds(i * L, L)] = zeros

            pl.loop(0, B)(zero_body)

        start = jnp.where(wid == NW - 1, N - CH, wid * CH)
        glo = jnp.where(wid == NW - 1, (CH - TAIL) // L, 0)
        Q = HALF // 2
        QG = Q // L
        c0 = pltpu.async_copy(x_hbm.at[pl.ds(start, Q)],
                              xa.at[pl.ds(0, Q)], sa)
        c1 = pltpu.async_copy(x_hbm.at[pl.ds(start + Q, Q)],
                              xa.at[pl.ds(Q, Q)], sb)
        c2 = pltpu.async_copy(x_hbm.at[pl.ds(start + 2 * Q, Q)],
                              xb.at[pl.ds(0, Q)], sa)
        c3 = pltpu.async_copy(x_hbm.at[pl.ds(start + 3 * Q, Q)],
                              xb.at[pl.ds(Q, Q)], sb)
        pltpu.sync_copy(b_hbm.at[pl.ds(start, CH)], bv)
        zero_acc()
        sglo = bv[pl.ds(glo * L, L)][0]
        carry = (sglo, zeros, [zeros for _ in range(NJ)])
        c0.wait()
        carry = chunk_groups(xa, 0, glo, QG, carry)
        c1.wait()
        carry = chunk_groups(xa, 0, jnp.maximum(glo, QG), 2 * QG, carry)
        c2.wait()
        carry = chunk_groups(xb, -2 * QG, jnp.maximum(glo, 2 * QG),
                             3 * QG, carry)
        c3.wait()
        carry = chunk_groups(xb, -2 * QG, jnp.maximum(glo, 3 * QG),
                             4 * QG, carry)
        flush(carry[0], carry[2], carry[1])

        p0 = pltpu.async_copy(acc, sh_sum.at[sid], sa)
        p1 = pltpu.async_copy(cnt, sh_cnt.at[sid], sb)
        p0.wait()
        p1.wait()
        plsc.subcore_barrier()
        cps = []
        for r in range(NS):
            cps.append(pltpu.async_copy(
                sh_sum.at[r, pl.ds(sid * SL, SL)], rsum.at[r], sa))
            cps.append(pltpu.async_copy(
                sh_cnt.at[r, pl.ds(sid * SLC, SLC)], rcnt.at[r], sb))
        for c in cps:
            c.wait()

        def red_sum(m):
            tot = rsum[0, pl.ds(m * L, L)]
            for r in range(1, NS):
                tot = tot + rsum[r, pl.ds(m * L, L)]
            acc[pl.ds(m * L, L)] = tot

        pl.loop(0, SL // L)(red_sum)

        for m in range(SLC // L):
            tot = rcnt[0, pl.ds(m * L, L)]
            for r in range(1, NS):
                tot = tot + rcnt[r, pl.ds(m * L, L)]
            cnt[pl.ds(m * L, L)] = tot

        RPT = B // NS
        wb = []
        for r in range(RPT):
            wb.append(pltpu.async_copy(
                acc.at[pl.ds(r * F, F)],
                psum_hbm.at[cid * B + sid * RPT + r], sa))
        wb.append(pltpu.async_copy(
            cnt.at[pl.ds(0, SLC)],
            pcnt_hbm.at[cid * 8 + sid // 2, pl.ds((sid % 2) * SLC, SLC)],
            sb))
        for c in wb:
            c.wait()

    return sc_kernel(x, batch)


def _tc_finish(psum, pcnt, u, W, b2):
    def tc_body(ps_ref, pc_ref, u_ref, w_ref, b_ref, out_ref):
        ps = ps_ref[...]
        sums = ps[:B] + ps[B:]
        ct = pc_ref[...]
        ctot = ct[:8] + ct[8:]
        si = jax.lax.broadcasted_iota(jnp.int32, (B, 8), 0)
        ri = jax.lax.broadcasted_iota(jnp.int32, (B, 8), 1)
        rsel = (ri == si // 8).astype(jnp.float32)
        rows = jnp.dot(rsel, ctot, preferred_element_type=jnp.float32)
        si2 = jax.lax.broadcasted_iota(jnp.int32, (B, F), 0)
        ci = jax.lax.broadcasted_iota(jnp.int32, (B, F), 1)
        csel = (ci == (si2 % 8) * L).astype(jnp.float32)
        counts = jnp.sum(rows * csel, axis=1, keepdims=True)
        x_agg = sums / jnp.maximum(counts, 1.0)
        w = w_ref[...]
        out = (
            jnp.dot(x_agg, w[:F], preferred_element_type=jnp.float32)
            + jnp.dot(u_ref[...], w[F:], preferred_element_type=jnp.float32)
            + b_ref[...]
        )
        out_ref[...] = out

    return pl.pallas_call(
        tc_body,
        out_shape=jax.ShapeDtypeStruct((B, B), jnp.float32),
    )(psum, pcnt, u, W, b2)


def kernel(x, edge_index, edge_attr, u, batch, W, b):
    psum, pcnt = _sc_segment_partials(x, batch)
    return _tc_finish(psum, pcnt, u, W, b.reshape(1, B))


# speedup vs baseline: 1.0655x; 1.0259x over previous
import functools

import jax
import jax.numpy as jnp
from jax import lax
from jax.experimental import pallas as pl
from jax.experimental.pallas import tpu as pltpu
from jax.experimental.pallas import tpu_sc as plsc

N = 10000
F = 128
B = 64
NC = 2
NS = 16
NW = NC * NS
L = 16
CH = 320
TAIL_START = 31 * CH
TAIL = N - TAIL_START
HALF = CH // 2
NJ = F // L


def _sc_segment_partials(x, batch):
    mesh = plsc.VectorSubcoreMesh(core_axis_name="c", subcore_axis_name="s")

    SL = B * F // NS
    SLC = B * L // NS

    @functools.partial(
        pl.kernel,
        out_type=[
            jax.ShapeDtypeStruct((NC * B, F), jnp.float32),
            jax.ShapeDtypeStruct((NC * B * L // F, F), jnp.float32),
        ],
        mesh=mesh,
        scratch_types=[
            pltpu.VMEM((HALF, F), jnp.float32),
            pltpu.VMEM((HALF, F), jnp.float32),
            pltpu.VMEM((CH,), jnp.int32),
            pltpu.VMEM((B * F,), jnp.float32),
            pltpu.VMEM((B * L,), jnp.float32),
            pltpu.VMEM((NS, SL), jnp.float32),
            pltpu.VMEM((NS, SLC), jnp.float32),
            pltpu.VMEM_SHARED((NS, B * F), jnp.float32),
            pltpu.VMEM_SHARED((NS, B * L), jnp.float32),
            pltpu.SemaphoreType.DMA,
            pltpu.SemaphoreType.DMA,
        ],
    )
    def sc_kernel(x_hbm, b_hbm, psum_hbm, pcnt_hbm, xa, xb, bv, acc, cnt,
                  rsum, rcnt, sh_sum, sh_cnt, sa, sb):
        sid = lax.axis_index("s")
        cid = lax.axis_index("c")
        wid = sid * NC + cid

        zeros = jnp.zeros((L,), jnp.float32)
        ones_v = jnp.ones((L,), jnp.float32)

        def flush(seg, accv, cntf):
            plsc.addupdate(cnt.at[pl.ds(seg * L, L)], cntf)
            for j in range(NJ):
                plsc.addupdate(acc.at[pl.ds(seg * F + j * L, L)], accv[j])

        def chunk_groups(xv, roff_g, lo, hi, carry):
            def grp_body(g, c):
                cur, cntf, accv = c
                segv = bv[pl.ds(g * L, L)]
                for k in range(L):
                    s = segv[k]
                    is_new = s != cur

                    @pl.when(is_new)
                    def _():
                        flush(cur, accv, cntf)

                    keep = jnp.where(is_new, 0.0, 1.0)
                    row = [xv[(g + roff_g) * L + k, pl.ds(j * L, L)]
                           for j in range(NJ)]
                    accv = [accv[j] * keep + row[j] for j in range(NJ)]
                    cntf = cntf * keep + ones_v
                    cur = s
                return (cur, cntf, accv)

            return pl.loop(lo, hi, init_carry=carry)(grp_body)

        def zero_acc():
            def zero_body(i):
                for j in range(NJ):
                    acc[pl.ds(i * F + j * L, L)] = zeros
                cnt[pl.ds(i * L, L)] = zeros

            pl.loop(0, B)(zero_body)

        start = jnp.where(wid == NW - 1, N - CH, wid * CH)
        glo = jnp.where(wid == NW - 1, (CH - TAIL) // L, 0)
        c0 = pltpu.async_copy(x_hbm.at[pl.ds(start, HALF)], xa, sa)
        c1 = pltpu.async_copy(x_hbm.at[pl.ds(start + HALF, HALF)], xb, sb)
        pltpu.sync_copy(b_hbm.at[pl.ds(start, CH)], bv)
        zero_acc()
        sglo = bv[pl.ds(glo * L, L)][0]
        carry = (sglo, zeros, [zeros for _ in range(NJ)])
        c0.wait()
        carry = chunk_groups(xa, 0, glo, HALF // L, carry)
        c1.wait()
        carry = chunk_groups(xb, -(HALF // L), jnp.maximum(glo, HALF // L),
                             CH // L, carry)
        flush(carry[0], carry[2], carry[1])

        p0 = pltpu.async_copy(acc, sh_sum.at[sid], sa)
        p1 = pltpu.async_copy(cnt, sh_cnt.at[sid], sb)
        p0.wait()
        p1.wait()
        plsc.subcore_barrier()
        cps = []
        for r in range(NS):
            cps.append(pltpu.async_copy(
                sh_sum.at[r, pl.ds(sid * SL, SL)], rsum.at[r], sa))
            cps.append(pltpu.async_copy(
                sh_cnt.at[r, pl.ds(sid * SLC, SLC)], rcnt.at[r], sb))
        for c in cps:
            c.wait()

        def red_sum(m):
            tot = rsum[0, pl.ds(m * L, L)]
            for r in range(1, NS):
                tot = tot + rsum[r, pl.ds(m * L, L)]
            acc[pl.ds(m * L, L)] = tot

        pl.loop(0, SL // L)(red_sum)

        for m in range(SLC // L):
            tot = rcnt[0, pl.ds(m * L, L)]
            for r in range(1, NS):
                tot = tot + rcnt[r, pl.ds(m * L, L)]
            cnt[pl.ds(m * L, L)] = tot

        RPT = B // NS
        wb = []
        for r in range(RPT):
            wb.append(pltpu.async_copy(
                acc.at[pl.ds(r * F, F)],
                psum_hbm.at[cid * B + sid * RPT + r], sa))
        wb.append(pltpu.async_copy(
            cnt.at[pl.ds(0, SLC)],
            pcnt_hbm.at[cid * 8 + sid // 2, pl.ds((sid % 2) * SLC, SLC)],
            sb))
        for c in wb:
            c.wait()

    return sc_kernel(x, batch)


def _tc_finish(psum, pcnt, u, W, b2):
    def tc_body(ps_ref, pc_ref, u_ref, w_ref, b_ref, out_ref):
        ps = ps_ref[...]
        sums = ps[:B] + ps[B:]
        ct = pc_ref[...]
        ctot = ct[:8] + ct[8:]
        si = jax.lax.broadcasted_iota(jnp.int32, (B, 8), 0)
        ri = jax.lax.broadcasted_iota(jnp.int32, (B, 8), 1)
        rsel = (ri == si // 8).astype(jnp.float32)
        rows = jnp.dot(rsel, ctot, preferred_element_type=jnp.float32)
        si2 = jax.lax.broadcasted_iota(jnp.int32, (B, F), 0)
        ci = jax.lax.broadcasted_iota(jnp.int32, (B, F), 1)
        csel = (ci == (si2 % 8) * L).astype(jnp.float32)
        counts = jnp.sum(rows * csel, axis=1, keepdims=True)
        x_agg = sums / jnp.maximum(counts, 1.0)
        w = w_ref[...]
        out = (
            jnp.dot(x_agg, w[:F], preferred_element_type=jnp.float32)
            + jnp.dot(u_ref[...], w[F:], preferred_element_type=jnp.float32)
            + b_ref[...]
        )
        out_ref[...] = out

    return pl.pallas_call(
        tc_body,
        out_shape=jax.ShapeDtypeStruct((B, B), jnp.float32),
    )(psum, pcnt, u, W, b2)


def kernel(x, edge_index, edge_attr, u, batch, W, b):
    psum, pcnt = _sc_segment_partials(x, batch)
    return _tc_finish(psum, pcnt, u, W, b.reshape(1, B))
